# Initial kernel scaffold; baseline (speedup 1.0000x reference)
import jax
import jax.numpy as jnp
from jax.experimental import pallas as pl


def kernel(source_features, target_features, edge_index, W_src, b_src, W_tgt, b_tgt, attn_proj, W_msg, b_msg, W_out, b_out):
    raise NotImplementedError("write your pallas kernel here")


# speedup vs baseline: 17.3970x; 17.3970x over previous
import functools

import jax
import jax.numpy as jnp
from jax import lax
from jax.experimental import pallas as pl
from jax.experimental.pallas import tpu as pltpu
from jax.experimental.pallas import tpu_sc as plsc

H = 4
HID = 128
HTOT = 512
D = 128
HEAD_DIM = 32
N = 10000
E = 320000
NW = 32
EPW = E // NW
EB = 80
NBLK = EPW // EB
ROWW = 144
NP = 10240
RPS = NP // 16
EBLK = 512
NEB = E // EBLK
RBLK = 400

_sc_mesh = plsc.VectorSubcoreMesh(core_axis_name="c", subcore_axis_name="s")


def _node_mm_body(x_ref, w_ref, b_ref, o_ref):
    o_ref[...] = jnp.dot(x_ref[...], w_ref[...],
                         preferred_element_type=jnp.float32) + b_ref[...]


def _node_msg(x, w, b2):
    return pl.pallas_call(
        _node_mm_body,
        grid=(N // RBLK,),
        in_specs=[
            pl.BlockSpec((RBLK, D), lambda i: (i, 0)),
            pl.BlockSpec((D, D), lambda i: (0, 0)),
            pl.BlockSpec((1, D), lambda i: (0, 0)),
        ],
        out_specs=pl.BlockSpec((RBLK, D), lambda i: (i, 0)),
        out_shape=jax.ShapeDtypeStruct((N, D), jnp.float32),
    )(x, w, b2)


@functools.partial(
    pl.kernel,
    mesh=_sc_mesh,
    out_type=[
        jax.ShapeDtypeStruct((E, D), jnp.float32),
        jax.ShapeDtypeStruct((E, D), jnp.float32),
    ],
    scratch_types=[
        pltpu.VMEM((EB,), jnp.int32),
        pltpu.VMEM((EB,), jnp.int32),
        pltpu.VMEM((EB, D), jnp.float32),
        pltpu.VMEM((EB, D), jnp.float32),
        pltpu.SemaphoreType.DMA,
        pltpu.SemaphoreType.DMA,
    ],
    compiler_params=pltpu.CompilerParams(needs_layout_passes=False, use_tc_tiling_on_sc=False),
)
def _gather_cells(src_hbm, dst_hbm, sfeat_hbm, tfeat_hbm,
                  osc_hbm, otc_hbm,
                  sidx_v, didx_v, srow_v, trow_v, sem_a, sem_b):
    wid = lax.axis_index("s") * 2 + lax.axis_index("c")
    base0 = wid * EPW

    def body(i, carry):
        base = base0 + i * EB
        pltpu.sync_copy(src_hbm.at[pl.ds(base, EB)], sidx_v)
        pltpu.sync_copy(dst_hbm.at[pl.ds(base, EB)], didx_v)
        cp_a = pltpu.async_copy(sfeat_hbm.at[sidx_v], srow_v, sem_a)
        cp_b = pltpu.async_copy(tfeat_hbm.at[didx_v], trow_v, sem_b)
        cp_a.wait()
        cp_b.wait()
        pltpu.sync_copy(srow_v, osc_hbm.at[pl.ds(base, EB)])
        pltpu.sync_copy(trow_v, otc_hbm.at[pl.ds(base, EB)])
        return carry

    lax.fori_loop(0, NBLK, body, 0)


def _logits_body(sc_ref, tc_ref, ws_ref, wt_ref, b_ref, wap_ref,
                 lg_ref, bm_ref):
    h = jnp.dot(sc_ref[...], ws_ref[...], preferred_element_type=jnp.float32)
    h = h + jnp.dot(tc_ref[...], wt_ref[...],
                    preferred_element_type=jnp.float32)
    h = h + b_ref[...]
    h = h * 0.5 * (1.0 + lax.erf(h * 0.7071067811865476))
    lg = jnp.dot(h, wap_ref[...], preferred_element_type=jnp.float32)
    lg_ref[...] = lg
    i = pl.program_id(0)
    cur = jnp.full((8, 128), jnp.max(lg), jnp.float32)

    @pl.when(i == 0)
    def _init():
        bm_ref[...] = cur

    @pl.when(i > 0)
    def _acc():
        bm_ref[...] = jnp.maximum(bm_ref[...], cur)

    @pl.when(i == NEB - 1)
    def _flood():
        bm_ref[...] = jnp.full((8, 128), jnp.max(bm_ref[...]), jnp.float32)


def _edge_logits(sc_cells, tc_cells, w_src, w_tgt, bsum2, wap):
    return pl.pallas_call(
        _logits_body,
        grid=(NEB,),
        in_specs=[
            pl.BlockSpec((EBLK, D), lambda i: (i, 0)),
            pl.BlockSpec((EBLK, D), lambda i: (i, 0)),
            pl.BlockSpec((D, HTOT), lambda i: (0, 0)),
            pl.BlockSpec((D, HTOT), lambda i: (0, 0)),
            pl.BlockSpec((1, HTOT), lambda i: (0, 0)),
            pl.BlockSpec((HTOT, H), lambda i: (0, 0)),
        ],
        out_specs=[
            pl.BlockSpec((EBLK, H), lambda i: (i, 0)),
            pl.BlockSpec((8, 128), lambda i: (0, 0)),
        ],
        out_shape=[
            jax.ShapeDtypeStruct((E, H), jnp.float32),
            jax.ShapeDtypeStruct((8, 128), jnp.float32),
        ],
    )(sc_cells, tc_cells, w_src, w_tgt, bsum2, wap)


@functools.partial(
    pl.kernel,
    mesh=_sc_mesh,
    out_type=jax.ShapeDtypeStruct((2, NP, D), jnp.float32),
    scratch_types=[
        pltpu.VMEM((EB,), jnp.int32),
        pltpu.VMEM((EB,), jnp.int32),
        pltpu.VMEM((EB * H,), jnp.float32),
        pltpu.VMEM((EB * H,), jnp.float32),
        pltpu.VMEM((EB, D), jnp.float32),
        pltpu.VMEM((EB, D), jnp.float32),
        pltpu.VMEM((16,), jnp.float32),
        pltpu.VMEM_SHARED((NP, D), jnp.float32),
        pltpu.SemaphoreType.DMA,
    ],
    compiler_params=pltpu.CompilerParams(needs_layout_passes=False, use_tc_tiling_on_sc=False),
)
def _edge_accum(src_hbm, dst_hbm, lg_hbm, cmax_hbm, msg_hbm, num_hbm,
                sidx_v, didx_v, lg_v, ex_v, msgrow_v, build_v, cmax_v,
                shared, sem):
    cid = lax.axis_index("c")
    sid = lax.axis_index("s")
    wid = sid * 2 + cid
    base0 = wid * EPW
    row0 = sid * RPS

    pltpu.sync_copy(cmax_hbm, cmax_v)
    cmax = cmax_v[...]

    zero16 = jnp.zeros((16,), jnp.float32)

    def z_body(e, carry):
        for j in range(D // 16):
            build_v[e, pl.ds(16 * j, 16)] = zero16
        return carry

    lax.fori_loop(0, EB, z_body, 0)
    for k in range(RPS // EB):
        pltpu.sync_copy(build_v, shared.at[pl.ds(row0 + k * EB, EB)])
    plsc.subcore_barrier()

    def blk_body(i, carry):
        base = base0 + i * EB
        pltpu.sync_copy(src_hbm.at[pl.ds(base, EB)], sidx_v)
        pltpu.sync_copy(dst_hbm.at[pl.ds(base, EB)], didx_v)
        pltpu.sync_copy(lg_hbm.at[pl.ds(base * H, EB * H)], lg_v)
        cp = pltpu.async_copy(msg_hbm.at[sidx_v], msgrow_v, sem)

        def ex_body(k, carry2):
            lv = lg_v[pl.ds(k * 16, 16)]
            ex_v[pl.ds(k * 16, 16)] = jnp.exp(lv - cmax)
            return carry2

        lax.fori_loop(0, EB * H // 16, ex_body, 0)
        cp.wait()

        def edge_body(e, carry2):
            for j in range(D // 16):
                hh = j // 2
                idx = jnp.full((16,), 0, jnp.int32) + (e * H + hh)
                w = plsc.load_gather(ex_v, [idx])
                mv = msgrow_v[e, pl.ds(16 * j, 16)]
                build_v[e, pl.ds(16 * j, 16)] = w * mv
            return carry2

        lax.fori_loop(0, EB, edge_body, 0)
        pltpu.sync_copy(build_v, shared.at[didx_v], add=True)
        return carry

    lax.fori_loop(0, NBLK, blk_body, 0)
    plsc.subcore_barrier()

    for k in range(RPS // EB):
        pltpu.sync_copy(shared.at[pl.ds(row0 + k * EB, EB)],
                        num_hbm.at[cid, pl.ds(row0 + k * EB, EB)])


@functools.partial(
    pl.kernel,
    mesh=_sc_mesh,
    out_type=jax.ShapeDtypeStruct((NW, NP * H // 16, 16), jnp.float32),
    scratch_types=[
        pltpu.VMEM((EB,), jnp.int32),
        pltpu.VMEM((EB * H,), jnp.float32),
        pltpu.VMEM((EB * H,), jnp.float32),
        pltpu.VMEM((16,), jnp.float32),
        pltpu.VMEM((NP * H // 16, 16), jnp.float32),
    ],
    compiler_params=pltpu.CompilerParams(needs_layout_passes=False, use_tc_tiling_on_sc=False),
)
def _den_accum(dst_hbm, lg_hbm, cmax_hbm, den_hbm,
               didx_v, lg_v, ex_v, cmax_v, den_v):
    cid = lax.axis_index("c")
    sid = lax.axis_index("s")
    wid = sid * 2 + cid
    base0 = wid * EPW

    pltpu.sync_copy(cmax_hbm, cmax_v)
    cmax = cmax_v[...]
    zero16 = jnp.zeros((16,), jnp.float32)

    def zd_body(r, carry):
        den_v[r] = zero16
        return carry

    lax.fori_loop(0, NP * H // 16, zd_body, 0)

    lane = lax.iota(jnp.int32, 16)
    minlane = jnp.minimum(lane, H - 1)

    def blk_body(i, carry):
        base = base0 + i * EB
        pltpu.sync_copy(dst_hbm.at[pl.ds(base, EB)], didx_v)
        pltpu.sync_copy(lg_hbm.at[pl.ds(base * H, EB * H)], lg_v)

        def ex_body(k, carry2):
            lv = lg_v[pl.ds(k * 16, 16)]
            ex_v[pl.ds(k * 16, 16)] = jnp.exp(lv - cmax)
            return carry2

        lax.fori_loop(0, EB * H // 16, ex_body, 0)

        def edge_body(e, carry2):
            eidx = e * H + minlane
            wv = plsc.load_gather(ex_v, [eidx])
            dstv = plsc.load_gather(didx_v,
                                    [jnp.full((16,), 0, jnp.int32) + e])
            plsc.addupdate_scatter(
                den_v,
                [dstv // 4, (dstv % 4) * H + minlane],
                wv, mask=lane < H)
            return carry2

        lax.fori_loop(0, EB, edge_body, 0)
        return carry

    lax.fori_loop(0, NBLK, blk_body, 0)
    pltpu.sync_copy(den_v, den_hbm.at[wid])


def _final_body(p_ref, dp_ref, s_ref, wo_ref, b_ref, o_ref):
    num = p_ref[0] + p_ref[1]
    den = jnp.sum(dp_ref[...], axis=0)
    inv = jnp.where(den > 0.0, 1.0 / den, 0.0)
    g = num * jnp.dot(inv, s_ref[...], preferred_element_type=jnp.float32)
    o_ref[...] = jnp.dot(g, wo_ref[...],
                         preferred_element_type=jnp.float32) + b_ref[...]


def _finalize(num_parts, den_all, sel, w_out, b2):
    return pl.pallas_call(
        _final_body,
        grid=(N // RBLK,),
        in_specs=[
            pl.BlockSpec((2, RBLK, D), lambda i: (0, i, 0)),
            pl.BlockSpec((NW, RBLK, H), lambda i: (0, i, 0)),
            pl.BlockSpec((H, D), lambda i: (0, 0)),
            pl.BlockSpec((D, D), lambda i: (0, 0)),
            pl.BlockSpec((1, D), lambda i: (0, 0)),
        ],
        out_specs=pl.BlockSpec((RBLK, D), lambda i: (i, 0)),
        out_shape=jax.ShapeDtypeStruct((N, D), jnp.float32),
    )(num_parts, den_all, sel, w_out, b2)


def kernel(source_features, target_features, edge_index, W_src, b_src,
           W_tgt, b_tgt, attn_proj, W_msg, b_msg, W_out, b_out):
    dst = edge_index[0]
    src = edge_index[1]
    bsum2 = (b_src + b_tgt).reshape(1, HTOT)
    eye = jnp.eye(H, dtype=jnp.float32)
    wap = (attn_proj[:, :, None] * eye[:, None, :]).reshape(HTOT, H)
    sel = jnp.repeat(eye, HEAD_DIM, axis=1)

    msg_tab = _node_msg(source_features, W_msg, b_msg.reshape(1, D))
    sc_cells, tc_cells = _gather_cells(src, dst, source_features,
                                       target_features)
    logits, bmax = _edge_logits(sc_cells, tc_cells, W_src, W_tgt, bsum2, wap)
    cvec = bmax.reshape(-1)[:16]
    num_parts = _edge_accum(src, dst, logits.reshape(-1), cvec, msg_tab)
    den_parts = _den_accum(dst, logits.reshape(-1), cvec)
    den_all = den_parts.reshape(NW, NP, H)
    return _finalize(num_parts, den_all, sel, W_out, b_out.reshape(1, D))
